# baseline (speedup 1.0000x reference)
import jax
import jax.numpy as jnp
from jax import lax
from jax.experimental import pallas as pl
from jax.experimental.pallas import tpu as pltpu
from jax.experimental.pallas import tpu_sc as plsc

V = 320000
D = 128
G = 10000
NC = 2
NS = 16
NW = NC * NS
R = 128
NBLK = V // R
NJ = 79
SPAN = 624

_DNUMS = lax.GatherDimensionNumbers(
    offset_dims=(), collapsed_slice_dims=(0,), start_index_map=(0,))


def _take16(x, idx):
    return lax.gather(x, idx[:, None], _DNUMS, (1,),
                      mode=lax.GatherScatterMode.PROMISE_IN_BOUNDS)


def _sc_body(h_hbm, b_hbm, w_hbm, bias_hbm, u_hbm, z_hbm,
             hbuf0, hbuf1, ibuf0, ibuf1, ebuf, ztile, wbuf, bbuf,
             acc, hsem0, hsem1, usem0, usem1, isem0, isem1):
    c = lax.axis_index("c")
    s = lax.axis_index("s")
    w = s * NC + c
    w_low = w < 4
    lo = jnp.where(w_low, 79 * w, 78 * w + 4)

    hbufs = [hbuf0, hbuf1]
    ibufs = [ibuf0, ibuf1]
    hsems = [hsem0, hsem1]
    usems = [usem0, usem1]
    isems = [isem0, isem1]

    pltpu.sync_copy(w_hbm, wbuf)
    pltpu.sync_copy(bias_hbm, bbuf)

    zeros16 = jnp.zeros((16,), jnp.float32)

    def _zero(t, carry):
        hbuf0[t // 8, pl.ds((t % 8) * 16, 16)] = zeros16
        return carry

    lax.fori_loop(0, R * (D // 16), _zero, 0)

    def _zeroz(t, carry):
        ztile[pl.ds(t * 16, 16)] = zeros16
        return carry

    lax.fori_loop(0, G // 16, _zeroz, 0)

    base = s * SPAN
    for q in range(4):
        pltpu.sync_copy(hbuf0, acc.at[pl.ds(base + q * R, R)])
    pltpu.sync_copy(hbuf0.at[pl.ds(0, SPAN - 4 * R)],
                    acc.at[pl.ds(base + 4 * R, SPAN - 4 * R)])

    @pl.when(s == NS - 1)
    def _():
        pltpu.sync_copy(hbuf0.at[pl.ds(0, 16)], acc.at[pl.ds(G - 16, 16)])

    plsc.subcore_barrier()

    wv = [wbuf[k] for k in range(8)]
    bv = bbuf[...]
    lane = lax.iota(jnp.int32, 16)
    mask0 = lane == 0
    xor_idx = [jnp.bitwise_xor(lane, m) for m in (8, 4, 2, 1)]

    def _splat_sum(p):
        for ix in xor_idx:
            p = p + _take16(p, ix)
        return p

    def _live(j):
        return (w_low & (j < NJ)) | (j < 78)

    def _start_in(j, ph):
        row0 = (lo + j) * R
        pltpu.async_copy(h_hbm.at[pl.ds(row0, R)], hbufs[ph], hsems[ph])
        pltpu.async_copy(b_hbm.at[pl.ds(row0, R)], ibufs[ph], isems[ph])

    _start_in(0, 0)

    def _phase(j, ph):
        @pl.when(_live(j))
        def _():
            hb = hbufs[ph]
            ib = ibufs[ph]
            po = ph ^ 1
            pltpu.make_async_copy(h_hbm.at[pl.ds(0, R)], hb,
                                  hsems[ph]).wait()
            pltpu.make_async_copy(b_hbm.at[pl.ds(0, R)], ib,
                                  isems[ph]).wait()

            def _dot(g, carry):
                d = jnp.zeros((16,), jnp.float32)
                for dr in range(16):
                    rr = g * 16 + dr
                    hk = [hb[rr, pl.ds(16 * k, 16)] for k in range(8)]
                    p0 = hk[0] * wv[0] + hk[1] * wv[1]
                    p1 = hk[2] * wv[2] + hk[3] * wv[3]
                    p2 = hk[4] * wv[4] + hk[5] * wv[5]
                    p3 = hk[6] * wv[6] + hk[7] * wv[7]
                    p = _splat_sum((p0 + p1) + (p2 + p3))
                    d = jnp.where(lane == dr, p, d)
                ebuf[pl.ds(g * 16, 16)] = jnp.exp(d + bv)
                return carry

            lax.fori_loop(0, 8, _dot, 0)

            def _row(g, carry):
                ev16 = ebuf[pl.ds(g * 16, 16)]
                for dr in range(16):
                    rr = g * 16 + dr
                    esp = _take16(ev16, jnp.full((16,), dr, jnp.int32))
                    for k in range(8):
                        hb[rr, pl.ds(16 * k, 16)] = (
                            esp * hb[rr, pl.ds(16 * k, 16)])
                return carry

            lax.fori_loop(0, 8, _row, 0)

            @pl.when(j >= 1)
            def _():
                pltpu.make_async_copy(hbufs[po], acc.at[ibufs[po]],
                                      usems[po]).wait()

            @pl.when(_live(j + 1) & (j + 1 < NJ))
            def _():
                _start_in(j + 1, po)

            pltpu.async_copy(hb, acc.at[ib], usems[ph], add=True)

            for g in range(8):
                ids = ib[pl.ds(g * 16, 16)]
                ev16 = ebuf[pl.ds(g * 16, 16)]
                csum = plsc.cumsum(ev16)
                idp = _take16(ids, jnp.maximum(lane - 1, 0))
                start = plsc.cummax(
                    jnp.where((lane == 0) | (ids != idp), lane, 0))
                prev = _take16(csum, jnp.maximum(start - 1, 0))
                seg = csum - jnp.where(start > 0, prev, 0.0)
                idn = _take16(ids, jnp.minimum(lane + 1, 15))
                last = (lane == 15) | (ids != idn)
                plsc.addupdate_scatter(ztile, [ids], seg, mask=last)

    def _iter(j2, carry):
        _phase(2 * j2, 0)
        _phase(2 * j2 + 1, 1)
        return carry

    lax.fori_loop(0, (NJ + 1) // 2, _iter, 0)

    @pl.when(w_low)
    def _():
        pltpu.make_async_copy(hbuf0, acc.at[ibuf0], usems[0]).wait()

    @pl.when(~w_low)
    def _():
        pltpu.make_async_copy(hbuf1, acc.at[ibuf1], usems[1]).wait()

    plsc.subcore_barrier()

    for q in range(4):
        pltpu.sync_copy(acc.at[pl.ds(base + q * R, R)],
                        u_hbm.at[c, pl.ds(base + q * R, R)])
    pltpu.sync_copy(acc.at[pl.ds(base + 4 * R, SPAN - 4 * R)],
                    u_hbm.at[c, pl.ds(base + 4 * R, SPAN - 4 * R)])

    @pl.when(s == NS - 1)
    def _():
        pltpu.sync_copy(acc.at[pl.ds(G - 16, 16)],
                        u_hbm.at[c, pl.ds(G - 16, 16)])

    pltpu.sync_copy(ztile, z_hbm.at[c, s])


def _combine_body(u_ref, z_ref, o_ref):
    u = u_ref[0] + u_ref[1]
    z = jnp.sum(z_ref[...], axis=(0, 1))[:, None]
    o_ref[...] = jnp.where(z > 0, u / z, 0.0)


def kernel(H, batch, W, b):
    batch32 = batch.astype(jnp.int32)
    wrow = W.reshape(8, 16).astype(jnp.float32)
    bvec = jnp.broadcast_to(b.reshape(1), (16,)).astype(jnp.float32)

    mesh = plsc.VectorSubcoreMesh(core_axis_name="c", subcore_axis_name="s",
                                  num_cores=NC, num_subcores=NS)
    u_part, z_part = pl.kernel(
        _sc_body,
        out_type=(jax.ShapeDtypeStruct((NC, G, D), jnp.float32),
                  jax.ShapeDtypeStruct((NC, NS, G), jnp.float32)),
        mesh=mesh,
        compiler_params=pltpu.CompilerParams(needs_layout_passes=False),
        scratch_types=[
            pltpu.VMEM((R, D), jnp.float32),
            pltpu.VMEM((R, D), jnp.float32),
            pltpu.VMEM((R,), jnp.int32),
            pltpu.VMEM((R,), jnp.int32),
            pltpu.VMEM((R,), jnp.float32),
            pltpu.VMEM((G,), jnp.float32),
            pltpu.VMEM((8, 16), jnp.float32),
            pltpu.VMEM((16,), jnp.float32),
            pltpu.VMEM_SHARED((G, D), jnp.float32),
            pltpu.SemaphoreType.DMA,
            pltpu.SemaphoreType.DMA,
            pltpu.SemaphoreType.DMA,
            pltpu.SemaphoreType.DMA,
            pltpu.SemaphoreType.DMA,
            pltpu.SemaphoreType.DMA,
        ],
    )(H, batch32, wrow, bvec)

    out = pl.pallas_call(
        _combine_body,
        out_shape=jax.ShapeDtypeStruct((G, D), jnp.float32),
    )(u_part, z_part)
    return out


# speedup vs baseline: 1.0694x; 1.0694x over previous
import jax
import jax.numpy as jnp
from jax import lax
from jax.experimental import pallas as pl
from jax.experimental.pallas import tpu as pltpu
from jax.experimental.pallas import tpu_sc as plsc

V = 320000
D = 128
G = 10000
NC = 2
NS = 16
NW = NC * NS
R = 128
NBLK = V // R
NJ = 79
SPAN = 624

_DNUMS = lax.GatherDimensionNumbers(
    offset_dims=(), collapsed_slice_dims=(0,), start_index_map=(0,))


def _take16(x, idx):
    return lax.gather(x, idx[:, None], _DNUMS, (1,),
                      mode=lax.GatherScatterMode.PROMISE_IN_BOUNDS)


def _sc_body(h_hbm, b_hbm, w_hbm, bias_hbm, u_hbm, z_hbm,
             hbuf0, hbuf1, ibuf0, ibuf1, ebuf, ztile, wbuf, bbuf,
             acc, hsem0, hsem1, usem0, usem1):
    c = lax.axis_index("c")
    s = lax.axis_index("s")
    w = s * NC + c
    w_low = w < 4
    lo = jnp.where(w_low, 79 * w, 78 * w + 4)

    hbufs = [hbuf0, hbuf1]
    ibufs = [ibuf0, ibuf1]
    hsems = [hsem0, hsem1]
    usems = [usem0, usem1]

    pltpu.sync_copy(w_hbm, wbuf)
    pltpu.sync_copy(bias_hbm, bbuf)

    zeros16 = jnp.zeros((16,), jnp.float32)

    def _zero(t, carry):
        hbuf0[t // 8, pl.ds((t % 8) * 16, 16)] = zeros16
        return carry

    lax.fori_loop(0, R * (D // 16), _zero, 0)

    def _zeroz(t, carry):
        ztile[pl.ds(t * 16, 16)] = zeros16
        return carry

    lax.fori_loop(0, G // 16, _zeroz, 0)

    base = s * SPAN
    for q in range(4):
        pltpu.sync_copy(hbuf0, acc.at[pl.ds(base + q * R, R)])
    pltpu.sync_copy(hbuf0.at[pl.ds(0, SPAN - 4 * R)],
                    acc.at[pl.ds(base + 4 * R, SPAN - 4 * R)])

    @pl.when(s == NS - 1)
    def _():
        pltpu.sync_copy(hbuf0.at[pl.ds(0, 16)], acc.at[pl.ds(G - 16, 16)])

    plsc.subcore_barrier()

    wv = [wbuf[k] for k in range(8)]
    bv = bbuf[...]
    lane = lax.iota(jnp.int32, 16)
    mask0 = lane == 0
    xor_idx = [jnp.bitwise_xor(lane, m) for m in (8, 4, 2, 1)]

    def _splat_sum(p):
        for ix in xor_idx:
            p = p + _take16(p, ix)
        return p

    def _live(j):
        return (w_low & (j < NJ)) | (j < 78)

    def _start_in(j, ph):
        row0 = (lo + j) * R
        pltpu.async_copy(h_hbm.at[pl.ds(row0, R)], hbufs[ph], hsems[ph])
        pltpu.sync_copy(b_hbm.at[pl.ds(row0, R)], ibufs[ph])

    _start_in(0, 0)

    def _phase(j, ph):
        @pl.when(_live(j))
        def _():
            hb = hbufs[ph]
            ib = ibufs[ph]
            po = ph ^ 1
            pltpu.make_async_copy(h_hbm.at[pl.ds(0, R)], hb,
                                  hsems[ph]).wait()

            def _dot(g, carry):
                e16 = jnp.zeros((16,), jnp.float32)
                for sub in range(4):
                    d = jnp.zeros((16,), jnp.float32)
                    hks = []
                    for dr in range(4):
                        rr = g * 16 + sub * 4 + dr
                        hk = [hb[rr, pl.ds(16 * k, 16)] for k in range(8)]
                        hks.append(hk)
                        p0 = hk[0] * wv[0] + hk[1] * wv[1]
                        p1 = hk[2] * wv[2] + hk[3] * wv[3]
                        p2 = hk[4] * wv[4] + hk[5] * wv[5]
                        p3 = hk[6] * wv[6] + hk[7] * wv[7]
                        p = _splat_sum((p0 + p1) + (p2 + p3))
                        d = jnp.where(lane == dr, p, d)
                    ev = jnp.exp(d + bv)
                    for dr in range(4):
                        rr = g * 16 + sub * 4 + dr
                        esp = _take16(ev, jnp.full((16,), dr, jnp.int32))
                        for k in range(8):
                            hb[rr, pl.ds(16 * k, 16)] = esp * hks[dr][k]
                    e16 = jnp.where(
                        jnp.right_shift(lane, 2) == sub,
                        _take16(ev, jnp.bitwise_and(lane, 3)), e16)
                ebuf[pl.ds(g * 16, 16)] = e16
                return carry

            lax.fori_loop(0, 8, _dot, 0)

            @pl.when(j >= 1)
            def _():
                pltpu.make_async_copy(hbufs[po], acc.at[ibufs[po]],
                                      usems[po]).wait()

            @pl.when(_live(j + 1) & (j + 1 < NJ))
            def _():
                _start_in(j + 1, po)

            pltpu.async_copy(hb, acc.at[ib], usems[ph], add=True)

            for g in range(8):
                ids = ib[pl.ds(g * 16, 16)]
                ev16 = ebuf[pl.ds(g * 16, 16)]
                csum = plsc.cumsum(ev16)
                idp = _take16(ids, jnp.maximum(lane - 1, 0))
                start = plsc.cummax(
                    jnp.where((lane == 0) | (ids != idp), lane, 0))
                prev = _take16(csum, jnp.maximum(start - 1, 0))
                seg = csum - jnp.where(start > 0, prev, 0.0)
                idn = _take16(ids, jnp.minimum(lane + 1, 15))
                last = (lane == 15) | (ids != idn)
                plsc.addupdate_scatter(ztile, [ids], seg, mask=last)

    def _iter(j2, carry):
        _phase(2 * j2, 0)
        _phase(2 * j2 + 1, 1)
        return carry

    lax.fori_loop(0, (NJ + 1) // 2, _iter, 0)

    @pl.when(w_low)
    def _():
        pltpu.make_async_copy(hbuf0, acc.at[ibuf0], usems[0]).wait()

    @pl.when(~w_low)
    def _():
        pltpu.make_async_copy(hbuf1, acc.at[ibuf1], usems[1]).wait()

    plsc.subcore_barrier()

    for q in range(4):
        pltpu.sync_copy(acc.at[pl.ds(base + q * R, R)],
                        u_hbm.at[c, pl.ds(base + q * R, R)])
    pltpu.sync_copy(acc.at[pl.ds(base + 4 * R, SPAN - 4 * R)],
                    u_hbm.at[c, pl.ds(base + 4 * R, SPAN - 4 * R)])

    @pl.when(s == NS - 1)
    def _():
        pltpu.sync_copy(acc.at[pl.ds(G - 16, 16)],
                        u_hbm.at[c, pl.ds(G - 16, 16)])

    pltpu.sync_copy(ztile, z_hbm.at[c, s])


def _combine_body(u_ref, z_ref, o_ref):
    u = u_ref[0] + u_ref[1]
    z = jnp.sum(z_ref[...], axis=(0, 1))[:, None]
    o_ref[...] = jnp.where(z > 0, u / z, 0.0)


def kernel(H, batch, W, b):
    batch32 = batch.astype(jnp.int32)
    wrow = W.reshape(8, 16).astype(jnp.float32)
    bvec = jnp.broadcast_to(b.reshape(1), (16,)).astype(jnp.float32)

    mesh = plsc.VectorSubcoreMesh(core_axis_name="c", subcore_axis_name="s",
                                  num_cores=NC, num_subcores=NS)
    u_part, z_part = pl.kernel(
        _sc_body,
        out_type=(jax.ShapeDtypeStruct((NC, G, D), jnp.float32),
                  jax.ShapeDtypeStruct((NC, NS, G), jnp.float32)),
        mesh=mesh,
        compiler_params=pltpu.CompilerParams(needs_layout_passes=False),
        scratch_types=[
            pltpu.VMEM((R, D), jnp.float32),
            pltpu.VMEM((R, D), jnp.float32),
            pltpu.VMEM((R,), jnp.int32),
            pltpu.VMEM((R,), jnp.int32),
            pltpu.VMEM((R,), jnp.float32),
            pltpu.VMEM((G,), jnp.float32),
            pltpu.VMEM((8, 16), jnp.float32),
            pltpu.VMEM((16,), jnp.float32),
            pltpu.VMEM_SHARED((G, D), jnp.float32),
            pltpu.SemaphoreType.DMA,
            pltpu.SemaphoreType.DMA,
            pltpu.SemaphoreType.DMA,
            pltpu.SemaphoreType.DMA,
        ],
    )(H, batch32, wrow, bvec)

    out = pl.pallas_call(
        _combine_body,
        out_shape=jax.ShapeDtypeStruct((G, D), jnp.float32),
    )(u_part, z_part)
    return out
